# baseline (device time: 12812 ns/iter reference)
import jax
import jax.numpy as jnp
from jax import lax
from jax.experimental import pallas as pl
from jax.experimental.pallas import tpu as pltpu

N_DEV = 16
N_CHUNK = 8


def kernel(x):
    m_per, n = x.shape
    m_chunk = m_per // N_CHUNK

    def body(x_hbm, out_ref, chunk_ref, local_ref, copy_sems,
             comm_ref, send_sems, recv_sems):
        my = lax.axis_index("i")

        barrier_sem = pltpu.get_barrier_semaphore()
        for d in range(1, N_DEV):
            tgt = lax.rem(my + d, N_DEV)
            pl.semaphore_signal(
                barrier_sem, inc=1,
                device_id=(tgt,), device_id_type=pl.DeviceIdType.MESH,
            )

        copies = []
        for c in range(N_CHUNK):
            cp = pltpu.make_async_copy(
                x_hbm.at[pl.ds(c * m_chunk, m_chunk), :],
                chunk_ref.at[c],
                copy_sems.at[c],
            )
            cp.start()
            copies.append(cp)

        acc = None
        for c in range(N_CHUNK):
            copies[c].wait()
            blk = jnp.max(chunk_ref[c], axis=0)
            acc = blk if acc is None else jnp.maximum(acc, blk)
        local_ref[0, :] = acc

        pl.semaphore_wait(barrier_sem, N_DEV - 1)

        rdmas = []
        for d in range(1, N_DEV):
            tgt = lax.rem(my + d, N_DEV)
            rdma = pltpu.make_async_remote_copy(
                src_ref=local_ref,
                dst_ref=comm_ref.at[d - 1],
                send_sem=send_sems.at[d - 1],
                recv_sem=recv_sems.at[d - 1],
                device_id=(tgt,),
                device_id_type=pl.DeviceIdType.MESH,
            )
            rdma.start()
            rdmas.append(rdma)

        for r in rdmas:
            r.wait_recv()
        out_ref[0, :] = jnp.maximum(
            local_ref[0, :], jnp.max(comm_ref[:, 0, :], axis=0)
        )
        for r in rdmas:
            r.wait_send()

    return pl.pallas_call(
        body,
        out_shape=jax.ShapeDtypeStruct((1, n), x.dtype),
        in_specs=[pl.BlockSpec(memory_space=pl.ANY)],
        out_specs=pl.BlockSpec(memory_space=pltpu.VMEM),
        scratch_shapes=[
            pltpu.VMEM((N_CHUNK, m_chunk, n), x.dtype),
            pltpu.VMEM((1, n), x.dtype),
            pltpu.SemaphoreType.DMA((N_CHUNK,)),
            pltpu.VMEM((N_DEV - 1, 1, n), x.dtype),
            pltpu.SemaphoreType.DMA((N_DEV - 1,)),
            pltpu.SemaphoreType.DMA((N_DEV - 1,)),
        ],
        compiler_params=pltpu.CompilerParams(collective_id=0),
    )(x)
